# baseline (device time: 97113 ns/iter reference)
import jax
import jax.numpy as jnp
from jax import lax
from jax.experimental import pallas as pl
from jax.experimental.pallas import tpu as pltpu

N_DEV = 4

M = 64
KX = 2048
H = 4096
NB = 512
N_IN = H // NB
N_OUT = KX // NB


def kernel(x, Win0, Wout0, Win1, Wout1, Win2, Wout2):
    bf16 = jnp.bfloat16
    f32 = jnp.float32

    def body(x_ref, win0, wout0, win1, wout1, win2, wout2, o_ref,
             xa, hb, pair, buf1, buf2, win_buf, wout_buf,
             win_sem, wout_sem, s1, r1, s2, r2):
        me = lax.axis_index("i")
        p1 = me ^ 1
        p2 = me ^ 2
        wins = [win0, win1, win2]
        wouts = [wout0, wout1, wout2]

        def win_dma(l, j, slot):
            return pltpu.make_async_copy(
                wins[l].at[:, pl.ds(j * NB, NB)],
                win_buf.at[slot],
                win_sem.at[slot],
            )

        def wout_dma(l, j):
            return pltpu.make_async_copy(
                wouts[l].at[:, pl.ds(j * NB, NB)],
                wout_buf.at[j],
                wout_sem.at[j],
            )

        xa[...] = x_ref[...].astype(bf16)

        for j in range(3):
            win_dma(0, j, j).start()

        for l in range(3):
            for j in range(N_IN):
                slot = j % 3
                win_dma(l, j, slot).wait()
                wb = win_buf[slot].astype(bf16)
                hb[:, pl.ds(j * NB, NB)] = jnp.dot(
                    xa[...], wb, preferred_element_type=f32
                ).astype(bf16)
                nxt = j + 3
                if nxt < N_IN:
                    win_dma(l, nxt, nxt % 3).start()

            rdma1 = pltpu.make_async_remote_copy(
                src_ref=hb, dst_ref=buf1, send_sem=s1, recv_sem=r1,
                device_id=(p1,), device_id_type=pl.DeviceIdType.MESH,
            )
            rdma1.start()
            for j in range(N_OUT):
                wout_dma(l, j).start()
            rdma1.wait()
            pair[...] = (
                hb[...].astype(f32) + buf1[...].astype(f32)
            ).astype(bf16)
            rdma2 = pltpu.make_async_remote_copy(
                src_ref=pair, dst_ref=buf2, send_sem=s2, recv_sem=r2,
                device_id=(p2,), device_id_type=pl.DeviceIdType.MESH,
            )
            rdma2.start()
            rdma2.wait()
            hb[...] = jnp.maximum(
                pair[...].astype(f32) + buf2[...].astype(f32), 0.0
            ).astype(bf16)

            for j in range(N_OUT):
                wout_dma(l, j).wait()
                wb = wout_buf[j].astype(bf16)
                blk = jnp.dot(hb[...], wb, preferred_element_type=f32)
                if l == 2:
                    o_ref[:, pl.ds(j * NB, NB)] = blk
                else:
                    xa[:, pl.ds(j * NB, NB)] = blk.astype(bf16)
                if l < 2 and j >= N_OUT - 3:
                    k = j - (N_OUT - 3)
                    win_dma(l + 1, k, k).start()

    in_specs = [
        pl.BlockSpec(memory_space=pltpu.VMEM),
            pl.BlockSpec(memory_space=pltpu.MemorySpace.HBM),
            pl.BlockSpec(memory_space=pltpu.MemorySpace.HBM),
            pl.BlockSpec(memory_space=pltpu.MemorySpace.HBM),
            pl.BlockSpec(memory_space=pltpu.MemorySpace.HBM),
            pl.BlockSpec(memory_space=pltpu.MemorySpace.HBM),
            pl.BlockSpec(memory_space=pltpu.MemorySpace.HBM),
    ]
    scratch_shapes = [
            pltpu.VMEM((M, KX), jnp.bfloat16),
            pltpu.VMEM((M, H), jnp.bfloat16),
            pltpu.VMEM((M, H), jnp.bfloat16),
            pltpu.VMEM((M, H), jnp.bfloat16),
            pltpu.VMEM((M, H), jnp.bfloat16),
            pltpu.VMEM((3, KX, NB), jnp.float32),
            pltpu.VMEM((N_OUT, H, NB), jnp.float32),
            pltpu.SemaphoreType.DMA((3,)),
            pltpu.SemaphoreType.DMA((N_OUT,)),
            pltpu.SemaphoreType.DMA,
            pltpu.SemaphoreType.DMA,
            pltpu.SemaphoreType.DMA,
            pltpu.SemaphoreType.DMA,
    ]

    return pl.pallas_call(
        body,
        in_specs=in_specs,
        out_specs=pl.BlockSpec(memory_space=pltpu.VMEM),
        scratch_shapes=scratch_shapes,
        compiler_params=pltpu.CompilerParams(vmem_limit_bytes=100 * 1024 * 1024),
        out_shape=jax.ShapeDtypeStruct((M, KX), jnp.float32),
    )(x, Win0, Wout0, Win1, Wout1, Win2, Wout2)
